# baseline (speedup 1.0000x reference)
import functools

import jax
import jax.numpy as jnp
from jax import lax
from jax.experimental import pallas as pl
from jax.experimental.pallas import tpu as pltpu
from jax.experimental.pallas import tpu_sc as plsc

D = 512
B = 16
NC, NS, L = 2, 16, 16
NW = NC * NS
CH = 64
DV = D // L


def _segment_sums_sc(objects, lengths):
    mesh = plsc.VectorSubcoreMesh(core_axis_name="c", subcore_axis_name="s")

    @functools.partial(
        pl.kernel,
        mesh=mesh,
        out_type=jax.ShapeDtypeStruct((NW, B, D), jnp.float32),
        scratch_types=[
            pltpu.VMEM((B + L,), jnp.int32),
            pltpu.VMEM((2, CH + 8, D), jnp.float32),
            pltpu.VMEM((B + 1, D), jnp.float32),
            pltpu.SemaphoreType.DMA((2,)),
        ],
    )
    def seg_sum(obj_hbm, len_hbm, out_hbm, lenv, buf, accv, sems):
        wid = lax.axis_index("s") * NC + lax.axis_index("c")

        pltpu.sync_copy(len_hbm, lenv.at[pl.ds(0, B)])

        def scal(i):
            return lenv[pl.ds(i, L)][0]

        lanes = lax.iota(jnp.int32, L)
        csum_v = jnp.zeros((L,), jnp.int32)
        for i in range(B):
            csum_v = csum_v + jnp.where(lanes >= i, scal(i), 0)
        total = csum_v[B - 1]
        lo = (wid * total) // NW
        hi = ((wid + 1) * total) // NW

        zero = jnp.zeros((L,), jnp.float32)

        def zero_body(s, _):
            for j in range(DV):
                accv[s, pl.ds(j * L, L)] = zero
            return 0

        lax.fori_loop(0, B + 1, zero_body, 0)

        abase = (lo // 8) * 8
        roff = lo - abase
        nrows_w = hi - lo
        nchunks = (nrows_w + CH - 1) // CH

        def copy_desc(i, slot):
            astart = pl.multiple_of(abase + i * CH, 8)
            return pltpu.make_async_copy(
                obj_hbm.at[pl.ds(astart, CH + 8)], buf.at[slot], sems.at[slot]
            )

        @pl.when(nchunks > 0)
        def _():
            copy_desc(0, 0).start()

        ends = [csum_v[s] for s in range(B)]

        def chunk_body(i, _):
            slot = lax.rem(i, 2)
            copy_desc(i, slot).wait()

            @pl.when(i + 1 < nchunks)
            def _():
                copy_desc(i + 1, 1 - slot).start()

            base_g = lo + i * CH

            def block_body(rb, c):
                g_vec = base_g + rb * L + lanes
                cs_vec = jnp.zeros((L,), jnp.int32)
                for s in range(B):
                    cs_vec = cs_vec + jnp.where(ends[s] <= g_vec, 1, 0)
                cs_vec = jnp.where(g_vec >= hi, B, cs_vec)
                base_row = roff + rb * L
                for rr in range(L):
                    cs = cs_vec[rr]
                    row = base_row + rr
                    for j in range(DV):
                        plsc.addupdate(
                            accv.at[cs, pl.ds(j * L, L)],
                            buf[slot, row, pl.ds(j * L, L)],
                        )
                return c

            return lax.fori_loop(0, CH // L, block_body, 0)

        lax.fori_loop(0, nchunks, chunk_body, 0)
        pltpu.sync_copy(accv.at[pl.ds(0, B)], out_hbm.at[wid])

    return seg_sum(objects, lengths)


def _head_tc(partials, lengths_f, W, b2, gamma2, beta2):
    def body(p_ref, len_ref, w_ref, b_ref, g_ref, be_ref, o_ref):
        sums = jnp.sum(p_ref[...], axis=0)
        lenf = len_ref[...]
        scale = jnp.where(lenf > 0, 1.0 / jnp.maximum(lenf, 1.0), 0.0)
        avg = sums * scale
        z = (
            lax.dot_general(
                avg,
                w_ref[...],
                (((1,), (1,)), ((), ())),
                preferred_element_type=jnp.float32,
            )
            + b_ref[...]
        )
        mean = jnp.mean(z, axis=0, keepdims=True)
        var = jnp.mean((z - mean) ** 2, axis=0, keepdims=True)
        o_ref[...] = g_ref[...] * (z - mean) * lax.rsqrt(var + 1e-5) + be_ref[...]

    return pl.pallas_call(
        body,
        out_shape=jax.ShapeDtypeStruct((B, D), jnp.float32),
    )(partials, lengths_f, W, b2, gamma2, beta2)


def kernel(objects_squares, lengths, W, b, gamma, beta):
    partials = _segment_sums_sc(objects_squares, lengths)
    return _head_tc(
        partials,
        lengths.astype(jnp.float32).reshape(B, 1),
        W,
        b.reshape(1, D),
        gamma.reshape(1, D),
        beta.reshape(1, D),
    )
